# baseline (speedup 1.0000x reference)
import jax
import jax.numpy as jnp
from jax.experimental import pallas as pl
from jax.experimental.pallas import tpu as pltpu

B, C, H, W, O = 32, 64, 64, 64, 128
KH = KW = 3
HP, WP = H - KH + 1, W - KW + 1
COLS = HP * W
HW = H * W
CK = C * KH * KW
CKE = CK + 8
NCORES = 2
BPC = B // NCORES


def _hebb_kernel(x_ref, wq_ref, y_ref, a_ref, s_ref, pt_scr):
    j = pl.program_id(1)

    @pl.when(j == 0)
    def _tail():
        rid = jax.lax.broadcasted_iota(jnp.int32, (8, COLS), 0)
        pt_scr[CK:CKE, :] = jnp.where(rid == 0, 1.0, 0.0).astype(jnp.bfloat16)

    for t in range(KH * KW):
        kh, kw = divmod(t, KW)
        off = kh * W + kw
        n = min(HW - off, COLS)
        pt_scr[t * C:(t + 1) * C, 0:n] = x_ref[:, off:off + n]
        if n < COLS:
            pt_scr[t * C:(t + 1) * C, n:COLS] = jnp.zeros((C, COLS - n),
                                                          jnp.bfloat16)

    wq = wq_ref[...]
    lid = jax.lax.broadcasted_iota(jnp.int32, (1, CKE), 1)
    wsq = jnp.where(lid < CK, wq * wq, 0.0)
    nrm = jnp.sqrt(jnp.sum(wsq, axis=1, keepdims=True))
    rn = jnp.where(nrm == 0.0, 1.0, 1.0 / nrm)
    wn = jnp.where(lid < CK, wq * rn, wq).astype(jnp.bfloat16)

    ptb = pt_scr[...]

    y = jnp.dot(wn, ptb, preferred_element_type=jnp.float32)

    m = jnp.max(y, axis=0, keepdims=True)
    e = jnp.exp(y - m)
    r = e / jnp.sum(e, axis=0, keepdims=True)

    cid = jax.lax.broadcasted_iota(jnp.int32, (1, COLS), 1)
    rm = jnp.where(cid % W < WP, r, 0.0)
    r2 = rm * rm

    s1 = jnp.sum(rm, axis=1, keepdims=True)
    s2 = jnp.sum(r2, axis=1, keepdims=True)
    a = jax.lax.dot_general(r2.astype(jnp.bfloat16), ptb,
                            (((1,), (1,)), ((), ())),
                            preferred_element_type=jnp.float32)

    for i in range(HP):
        y_ref[:, i * WP:(i + 1) * WP] = y[:, i * W:i * W + WP]

    sj = jnp.concatenate([s1, s2], axis=1)

    @pl.when(j == 0)
    def _init():
        a_ref[...] = a
        s_ref[...] = sj

    @pl.when(j > 0)
    def _acc():
        a_ref[...] += a
        s_ref[...] += sj


@jax.jit
def kernel(x, weight, bias):
    xv = x.reshape(B, C, HW).astype(jnp.bfloat16)
    wflat = weight.transpose(0, 2, 3, 1).reshape(O, CK)
    wq = jnp.concatenate(
        [wflat, bias.reshape(O, 1), jnp.zeros((O, 7), jnp.float32)], axis=1)

    y_flat, a_part, s_part = pl.pallas_call(
        _hebb_kernel,
        grid=(NCORES, BPC),
        in_specs=[
            pl.BlockSpec((None, C, HW), lambda i, j: (i * BPC + j, 0, 0)),
            pl.BlockSpec((O, CKE), lambda i, j: (0, 0)),
        ],
        out_specs=[
            pl.BlockSpec((None, O, HP * WP), lambda i, j: (i * BPC + j, 0, 0)),
            pl.BlockSpec((None, O, CKE), lambda i, j: (i, 0, 0)),
            pl.BlockSpec((None, O, 2), lambda i, j: (i, 0, 0)),
        ],
        out_shape=[
            jax.ShapeDtypeStruct((B, O, HP * WP), jnp.float32),
            jax.ShapeDtypeStruct((NCORES, O, CKE), jnp.float32),
            jax.ShapeDtypeStruct((NCORES, O, 2), jnp.float32),
        ],
        scratch_shapes=[pltpu.VMEM((CKE, COLS), jnp.bfloat16)],
        compiler_params=pltpu.CompilerParams(
            dimension_semantics=("parallel", "arbitrary"),
            vmem_limit_bytes=56 * 1024 * 1024,
        ),
    )(xv, wq)

    y = y_flat.reshape(B, O, HP, WP)

    a = a_part.sum(axis=0)[:, :CK]
    s1 = s_part[:, :, 0].sum(axis=0)
    s2 = s_part[:, :, 1].sum(axis=0)
    r_sum = jnp.where(s1 == 0.0, 1.0, s1)
    a = a.reshape(O, KH, KW, C).transpose(0, 3, 1, 2)
    scale = (1.0 / r_sum)[:, None, None, None]
    delta_w = a * scale - (s2[:, None, None, None] * scale) * weight
    return y, delta_w


# speedup vs baseline: 1.0671x; 1.0671x over previous
import jax
import jax.numpy as jnp
from jax.experimental import pallas as pl
from jax.experimental.pallas import tpu as pltpu

B, C, H, W, O = 32, 64, 64, 64, 128
KH = KW = 3
HP, WP = H - KH + 1, W - KW + 1
COLS = HP * W
HW = H * W
CK = C * KH * KW
CKE = CK + 8
NCORES = 2
BPC = B // NCORES


def _hebb_kernel(x_ref, wq_ref, y_ref, a_ref, s_ref, pt_scr):
    j = pl.program_id(1)

    @pl.when(j == 0)
    def _tail():
        rid = jax.lax.broadcasted_iota(jnp.int32, (8, COLS), 0)
        pt_scr[CK:CKE, :] = jnp.where(rid == 0, 1.0, 0.0).astype(jnp.bfloat16)

    for t in range(KH * KW):
        kh, kw = divmod(t, KW)
        off = kh * W + kw
        n = min(HW - off, COLS)
        pt_scr[t * C:(t + 1) * C, 0:n] = x_ref[:, off:off + n].astype(
            jnp.bfloat16)
        if n < COLS:
            pt_scr[t * C:(t + 1) * C, n:COLS] = jnp.zeros((C, COLS - n),
                                                          jnp.bfloat16)

    wq = wq_ref[...]
    lid = jax.lax.broadcasted_iota(jnp.int32, (1, CKE), 1)
    wsq = jnp.where(lid < CK, wq * wq, 0.0)
    nrm = jnp.sqrt(jnp.sum(wsq, axis=1, keepdims=True))
    rn = jnp.where(nrm == 0.0, 1.0, 1.0 / nrm)
    wn = jnp.where(lid < CK, wq * rn, wq).astype(jnp.bfloat16)

    ptb = pt_scr[...]

    y = jnp.dot(wn, ptb, preferred_element_type=jnp.float32)

    m = jnp.max(y, axis=0, keepdims=True)
    e = jnp.exp(y - m)
    ssum = jnp.sum(e, axis=0, keepdims=True)
    cid = jax.lax.broadcasted_iota(jnp.int32, (1, COLS), 1)
    inv = jnp.where(cid % W < WP, 1.0 / ssum, 0.0)
    rmb = (e * inv).astype(jnp.bfloat16)
    r2b = rmb * rmb

    a = jax.lax.dot_general(r2b, ptb, (((1,), (1,)), ((), ())),
                            preferred_element_type=jnp.float32)
    s1m = jax.lax.dot_general(rmb, pt_scr[CK:CKE, :],
                              (((1,), (1,)), ((), ())),
                              preferred_element_type=jnp.float32)

    for i in range(HP):
        y_ref[:, i * WP:(i + 1) * WP] = y[:, i * W:i * W + WP]

    @pl.when(j == 0)
    def _init():
        a_ref[...] = a
        s_ref[...] = s1m

    @pl.when(j > 0)
    def _acc():
        a_ref[...] += a
        s_ref[...] += s1m


@jax.jit
def kernel(x, weight, bias):
    xv = x.reshape(B, C, HW)
    wflat = weight.transpose(0, 2, 3, 1).reshape(O, CK)
    wq = jnp.concatenate(
        [wflat, bias.reshape(O, 1), jnp.zeros((O, 7), jnp.float32)], axis=1)

    y_flat, a_part, s_part = pl.pallas_call(
        _hebb_kernel,
        grid=(NCORES, BPC),
        in_specs=[
            pl.BlockSpec((None, C, HW), lambda i, j: (i * BPC + j, 0, 0)),
            pl.BlockSpec((O, CKE), lambda i, j: (0, 0)),
        ],
        out_specs=[
            pl.BlockSpec((None, O, HP * WP), lambda i, j: (i * BPC + j, 0, 0)),
            pl.BlockSpec((None, O, CKE), lambda i, j: (i, 0, 0)),
            pl.BlockSpec((None, O, 8), lambda i, j: (i, 0, 0)),
        ],
        out_shape=[
            jax.ShapeDtypeStruct((B, O, HP * WP), jnp.float32),
            jax.ShapeDtypeStruct((NCORES, O, CKE), jnp.float32),
            jax.ShapeDtypeStruct((NCORES, O, 8), jnp.float32),
        ],
        scratch_shapes=[pltpu.VMEM((CKE, COLS), jnp.bfloat16)],
        compiler_params=pltpu.CompilerParams(
            dimension_semantics=("parallel", "arbitrary"),
            vmem_limit_bytes=56 * 1024 * 1024,
        ),
    )(xv, wq)

    y = y_flat.reshape(B, O, HP, WP)

    a_full = a_part.sum(axis=0)
    a = a_full[:, :CK]
    s1 = s_part[:, :, 0].sum(axis=0)
    s2 = a_full[:, CK]
    r_sum = jnp.where(s1 == 0.0, 1.0, s1)
    a = a.reshape(O, KH, KW, C).transpose(0, 3, 1, 2)
    scale = (1.0 / r_sum)[:, None, None, None]
    delta_w = a * scale - (s2[:, None, None, None] * scale) * weight
    return y, delta_w


# speedup vs baseline: 1.0996x; 1.0305x over previous
import jax
import jax.numpy as jnp
from jax.experimental import pallas as pl
from jax.experimental.pallas import tpu as pltpu

B, C, H, W, O = 32, 64, 64, 64, 128
KH = KW = 3
HP, WP = H - KH + 1, W - KW + 1
COLS = HP * W
HW = H * W
CK = C * KH * KW
CKE = CK + 8
NCORES = 2
BPC = B // NCORES


def _hebb_kernel(x_ref, wq_ref, y_ref, a_ref, s_ref, pt_scr):
    j = pl.program_id(1)

    @pl.when(j == 0)
    def _tail():
        rid = jax.lax.broadcasted_iota(jnp.int32, (8, COLS), 0)
        pt_scr[CK:CKE, :] = jnp.where(rid == 0, 1.0, 0.0).astype(jnp.bfloat16)

    for t in range(KH * KW):
        kh, kw = divmod(t, KW)
        off = kh * W + kw
        n = min(HW - off, COLS)
        pt_scr[t * C:(t + 1) * C, 0:n] = x_ref[:, off:off + n].astype(
            jnp.bfloat16)
        if n < COLS:
            pt_scr[t * C:(t + 1) * C, n:COLS] = jnp.zeros((C, COLS - n),
                                                          jnp.bfloat16)

    wq = wq_ref[...]
    lid = jax.lax.broadcasted_iota(jnp.int32, (1, CKE), 1)
    wsq = jnp.where(lid < CK, wq * wq, 0.0)
    nrm = jnp.sqrt(jnp.sum(wsq, axis=1, keepdims=True))
    rn = jnp.where(nrm == 0.0, 1.0, 1.0 / nrm)
    wn = jnp.where(lid < CK, wq * rn, wq).astype(jnp.bfloat16)

    ptb = pt_scr[...]

    y = jnp.dot(wn, ptb, preferred_element_type=jnp.float32)

    m = jnp.max(y, axis=0, keepdims=True)
    e = jnp.exp(y - m)
    ssum = jnp.sum(e, axis=0, keepdims=True)
    cid = jax.lax.broadcasted_iota(jnp.int32, (1, COLS), 1)
    inv = jnp.where(cid % W < WP, 1.0 / ssum, 0.0)
    rm = e * inv
    r2 = rm * rm

    s1 = jnp.sum(rm, axis=1, keepdims=True)
    a = jax.lax.dot_general(r2.astype(jnp.bfloat16), ptb,
                            (((1,), (1,)), ((), ())),
                            preferred_element_type=jnp.float32)
    s1m = jnp.concatenate([s1] * 8, axis=1)

    for i in range(HP):
        y_ref[:, i * WP:(i + 1) * WP] = y[:, i * W:i * W + WP]

    @pl.when(j == 0)
    def _init():
        a_ref[...] = a
        s_ref[...] = s1m

    @pl.when(j > 0)
    def _acc():
        a_ref[...] += a
        s_ref[...] += s1m


@jax.jit
def kernel(x, weight, bias):
    xv = x.reshape(B, C, HW)
    wflat = weight.transpose(0, 2, 3, 1).reshape(O, CK)
    wq = jnp.concatenate(
        [wflat, bias.reshape(O, 1), jnp.zeros((O, 7), jnp.float32)], axis=1)

    y_flat, a_part, s_part = pl.pallas_call(
        _hebb_kernel,
        grid=(NCORES, BPC),
        in_specs=[
            pl.BlockSpec((None, C, HW), lambda i, j: (i * BPC + j, 0, 0)),
            pl.BlockSpec((O, CKE), lambda i, j: (0, 0)),
        ],
        out_specs=[
            pl.BlockSpec((None, O, HP * WP), lambda i, j: (i * BPC + j, 0, 0)),
            pl.BlockSpec((None, O, CKE), lambda i, j: (i, 0, 0)),
            pl.BlockSpec((None, O, 8), lambda i, j: (i, 0, 0)),
        ],
        out_shape=[
            jax.ShapeDtypeStruct((B, O, HP * WP), jnp.float32),
            jax.ShapeDtypeStruct((NCORES, O, CKE), jnp.float32),
            jax.ShapeDtypeStruct((NCORES, O, 8), jnp.float32),
        ],
        scratch_shapes=[pltpu.VMEM((CKE, COLS), jnp.bfloat16)],
        compiler_params=pltpu.CompilerParams(
            dimension_semantics=("parallel", "arbitrary"),
            vmem_limit_bytes=56 * 1024 * 1024,
        ),
    )(xv, wq)

    y = y_flat.reshape(B, O, HP, WP)

    a_full = a_part.sum(axis=0)
    a = a_full[:, :CK]
    s1 = s_part[:, :, 0].sum(axis=0)
    s2 = a_full[:, CK]
    r_sum = jnp.where(s1 == 0.0, 1.0, s1)
    a = a.reshape(O, KH, KW, C).transpose(0, 3, 1, 2)
    scale = (1.0 / r_sum)[:, None, None, None]
    delta_w = a * scale - (s2[:, None, None, None] * scale) * weight
    return y, delta_w


# speedup vs baseline: 1.1245x; 1.0227x over previous
import jax
import jax.numpy as jnp
from jax.experimental import pallas as pl
from jax.experimental.pallas import tpu as pltpu

B, C, H, W, O = 32, 64, 64, 64, 128
KH = KW = 3
HP, WP = H - KH + 1, W - KW + 1
COLS = HP * W
HW = H * W
CK = C * KH * KW
CKE = CK + 8
NCORES = 2
BPC = B // NCORES


def _hebb_kernel(x_ref, wq_ref, y_ref, a_ref, s_ref, pt_scr):
    j = pl.program_id(1)

    @pl.when(j == 0)
    def _tail():
        rid = jax.lax.broadcasted_iota(jnp.int32, (8, COLS), 0)
        pt_scr[CK:CKE, :] = jnp.where(rid == 0, 1.0, 0.0).astype(jnp.bfloat16)

    for t in range(KH * KW):
        kh, kw = divmod(t, KW)
        off = kh * W + kw
        n = min(HW - off, COLS)
        pt_scr[t * C:(t + 1) * C, 0:n] = x_ref[:, off:off + n].astype(
            jnp.bfloat16)
        if n < COLS:
            pt_scr[t * C:(t + 1) * C, n:COLS] = jnp.zeros((C, COLS - n),
                                                          jnp.bfloat16)

    wq = wq_ref[...]
    lid = jax.lax.broadcasted_iota(jnp.int32, (1, CKE), 1)
    wsq = jnp.where(lid < CK, wq * wq, 0.0)
    nrm = jnp.sqrt(jnp.sum(wsq, axis=1, keepdims=True))
    rn = jnp.where(nrm == 0.0, 1.0, 1.0 / nrm)
    wn = jnp.where(lid < CK, wq * rn, wq).astype(jnp.bfloat16)

    ptb = pt_scr[...]

    y = jnp.dot(wn, ptb, preferred_element_type=jnp.float32)

    m = jnp.max(y, axis=0, keepdims=True)
    e = jnp.exp(y - m)
    ssum = jnp.sum(e, axis=0, keepdims=True)
    cid = jax.lax.broadcasted_iota(jnp.int32, (1, COLS), 1)
    inv = jnp.where(cid % W < WP, 1.0 / ssum, 0.0)
    rm = e * inv
    r2 = rm * rm

    s1 = jnp.sum(rm, axis=1, keepdims=True)
    a = jax.lax.dot_general(r2.astype(jnp.bfloat16), ptb,
                            (((1,), (1,)), ((), ())),
                            preferred_element_type=jnp.float32)
    s1m = jnp.concatenate([s1] * 8, axis=1)

    yb = y.astype(jnp.bfloat16)
    for i in range(HP):
        y_ref[:, i * WP:(i + 1) * WP] = yb[:, i * W:i * W + WP]

    @pl.when(j == 0)
    def _init():
        a_ref[...] = a
        s_ref[...] = s1m

    @pl.when(j > 0)
    def _acc():
        a_ref[...] += a
        s_ref[...] += s1m


@jax.jit
def kernel(x, weight, bias):
    xv = x.reshape(B, C, HW)
    wflat = weight.transpose(0, 2, 3, 1).reshape(O, CK)
    wq = jnp.concatenate(
        [wflat, bias.reshape(O, 1), jnp.zeros((O, 7), jnp.float32)], axis=1)

    y_flat, a_part, s_part = pl.pallas_call(
        _hebb_kernel,
        grid=(NCORES, BPC),
        in_specs=[
            pl.BlockSpec((None, C, HW), lambda i, j: (i * BPC + j, 0, 0)),
            pl.BlockSpec((O, CKE), lambda i, j: (0, 0)),
        ],
        out_specs=[
            pl.BlockSpec((None, O, HP * WP), lambda i, j: (i * BPC + j, 0, 0)),
            pl.BlockSpec((None, O, CKE), lambda i, j: (i, 0, 0)),
            pl.BlockSpec((None, O, 8), lambda i, j: (i, 0, 0)),
        ],
        out_shape=[
            jax.ShapeDtypeStruct((B, O, HP * WP), jnp.bfloat16),
            jax.ShapeDtypeStruct((NCORES, O, CKE), jnp.float32),
            jax.ShapeDtypeStruct((NCORES, O, 8), jnp.float32),
        ],
        scratch_shapes=[pltpu.VMEM((CKE, COLS), jnp.bfloat16)],
        compiler_params=pltpu.CompilerParams(
            dimension_semantics=("parallel", "arbitrary"),
            vmem_limit_bytes=56 * 1024 * 1024,
        ),
    )(xv, wq)

    y = y_flat.reshape(B, O, HP, WP).astype(jnp.float32)

    a_full = a_part.sum(axis=0)
    a = a_full[:, :CK]
    s1 = s_part[:, :, 0].sum(axis=0)
    s2 = a_full[:, CK]
    r_sum = jnp.where(s1 == 0.0, 1.0, s1)
    a = a.reshape(O, KH, KW, C).transpose(0, 3, 1, 2)
    scale = (1.0 / r_sum)[:, None, None, None]
    delta_w = a * scale - (s2[:, None, None, None] * scale) * weight
    return y, delta_w
